# Initial kernel scaffold; baseline (speedup 1.0000x reference)
import jax
import jax.numpy as jnp
from jax.experimental import pallas as pl


def kernel(x, emb_table, pos_table):
    raise NotImplementedError("write your pallas kernel here")


# speedup vs baseline: 3.4463x; 3.4463x over previous
import functools

import jax
import jax.numpy as jnp
from jax import lax
from jax.experimental import pallas as pl
from jax.experimental.pallas import tpu as pltpu
from jax.experimental.pallas import tpu_sc as plsc


@functools.lru_cache(maxsize=None)
def _build(B, S, H, V):
    info = plsc.get_sparse_core_info()
    NC, NS = info.num_cores, info.num_subcores
    NW = NC * NS
    N = B * S
    SEQ_PER_CHUNK = 2
    CH = SEQ_PER_CHUNK * S
    G = 4
    assert CH % G == 0
    GSZ = CH // G
    assert N % (NW * CH) == 0
    NCHUNK = N // (NW * CH)
    NT = NW * NCHUNK
    LANES = 16
    assert H % LANES == 0

    mesh = plsc.VectorSubcoreMesh(core_axis_name="c", subcore_axis_name="s")

    @functools.partial(
        pl.kernel,
        out_type=jax.ShapeDtypeStruct((NT, CH, H), jnp.float32),
        mesh=mesh,
        compiler_params=pltpu.CompilerParams(use_tc_tiling_on_sc=False),
        scratch_types=[
            pltpu.VMEM((G, GSZ), jnp.int32),
            pltpu.VMEM((CH, H), jnp.float32),
            pltpu.VMEM((S, H), jnp.float32),
            pltpu.SemaphoreType.DMA,
        ],
    )
    def k(x_hbm, emb_hbm, pos_hbm, out_hbm, idx_v, rows_v, pos_v, sem):
        wid = lax.axis_index("s") * NC + lax.axis_index("c")
        pltpu.sync_copy(pos_hbm, pos_v)

        def chunk_body(c, carry):
            t = wid * NCHUNK + c
            pltpu.sync_copy(x_hbm.at[t], idx_v)
            cps = [
                pltpu.make_async_copy(
                    emb_hbm.at[idx_v.at[g]],
                    rows_v.at[pl.ds(g * GSZ, GSZ)],
                    sem,
                )
                for g in range(G)
            ]
            for cp in cps:
                cp.start()
            for cp in cps:
                cp.wait()

            def pos_body(p, carry2):
                for j in range(H // LANES):
                    pv = pos_v[p, pl.ds(j * LANES, LANES)]
                    for s_ in range(SEQ_PER_CHUNK):
                        r = s_ * S + p
                        rows_v[r, pl.ds(j * LANES, LANES)] = (
                            rows_v[r, pl.ds(j * LANES, LANES)] + pv
                        )
                return carry2

            lax.fori_loop(0, S, pos_body, 0)
            pltpu.sync_copy(rows_v, out_hbm.at[t])
            return carry

        lax.fori_loop(0, NCHUNK, chunk_body, 0)

    return k, NT, CH


def kernel(x, emb_table, pos_table):
    B, S = x.shape
    V, H = emb_table.shape
    k, NT, CH = _build(B, S, H, V)
    G = 4
    x_r = x.astype(jnp.int32).reshape(NT, G, CH // G)
    out = k(x_r, emb_table, pos_table)
    return out.reshape(B, S, H)
